# baseline (speedup 1.0000x reference)
import functools

import jax
import jax.numpy as jnp
from jax import lax
from jax.experimental import pallas as pl
from jax.experimental.pallas import tpu as pltpu
from jax.experimental.pallas import tpu_sc as plsc

B, N, W, DIM, H, NPROC = 32, 256, 16, 64, 32, 4
E_ORG = 16384
D = N * W
CBLK = 256
C = D // CBLK
CH = 1024
NCH = B * N // CH
NSTEP = B + 4 * NCH


def _ae_body(x_ref, w_ref, b_ref, g_ref, bt_ref, o_ref, zmid):
    l = pl.program_id(0)
    c = pl.program_id(1)

    def layer(zin):
        h = jax.lax.dot_general(zin, w_ref[0], (((1,), (1,)), ((), ())),
                                preferred_element_type=jnp.float32)
        h = h + b_ref[0, 0]
        m = jnp.mean(h, axis=0, keepdims=True)
        v = jnp.mean((h - m) * (h - m), axis=0, keepdims=True)
        return jax.nn.sigmoid(
            (h - m) / jnp.sqrt(v + 1e-5) * g_ref[0, 0] + bt_ref[0, 0])

    @pl.when(l == 0)
    def _():
        zb = layer(x_ref[...])
        zmid[:, pl.ds(c * CBLK, CBLK)] = zb
        o_ref[...] = zb

    @pl.when(l == 1)
    def _():
        o_ref[...] = layer(zmid[...])


def _ae(x, Ws, bs, gs, bts):
    return pl.pallas_call(
        _ae_body,
        grid=(2, C),
        in_specs=[
            pl.BlockSpec((B, D), lambda l, c: (0, 0)),
            pl.BlockSpec((1, CBLK, D), lambda l, c: (l, c, 0)),
            pl.BlockSpec((1, 1, 1, CBLK), lambda l, c: (l, c, 0, 0)),
            pl.BlockSpec((1, 1, 1, CBLK), lambda l, c: (l, c, 0, 0)),
            pl.BlockSpec((1, 1, 1, CBLK), lambda l, c: (l, c, 0, 0)),
        ],
        out_specs=pl.BlockSpec((B, CBLK), lambda l, c: (0, c)),
        out_shape=jax.ShapeDtypeStruct((B, D), jnp.float32),
        scratch_shapes=[pltpu.VMEM((B, D), jnp.float32)],
    )(x, Ws, bs.reshape(2, C, 1, CBLK), gs.reshape(2, C, 1, CBLK),
      bts.reshape(2, C, 1, CBLK))


EPART = 4
RG = 8
RPW = N // RG
ESL = E_ORG // EPART


def _build_A(src, dst, zeros):
    info = plsc.get_sparse_core_info()
    mesh = plsc.VectorSubcoreMesh(core_axis_name="c", subcore_axis_name="s")

    @functools.partial(
        pl.kernel, mesh=mesh,
        compiler_params=pltpu.CompilerParams(needs_layout_passes=False),
        out_type=jax.ShapeDtypeStruct((EPART, N * N), jnp.float32),
        scratch_types=[
            pltpu.VMEM((ESL,), jnp.int32),
            pltpu.VMEM((ESL,), jnp.int32),
            pltpu.VMEM((RPW * N,), jnp.float32),
        ],
    )
    def k(src_hbm, dst_hbm, z_hbm, a_hbm, src_v, dst_v, slab_v):
        wid = lax.axis_index("s") * info.num_cores + lax.axis_index("c")
        g = wid // EPART
        ep = wid % EPART
        lo = g * RPW
        pltpu.sync_copy(src_hbm.at[pl.ds(ep * ESL, ESL)], src_v)
        pltpu.sync_copy(dst_hbm.at[pl.ds(ep * ESL, ESL)], dst_v)
        pltpu.sync_copy(z_hbm, slab_v)

        ones = jnp.ones((16,), jnp.float32)

        def ebody(i, carry):
            d16 = dst_v[pl.ds(i * 16, 16)]
            s16 = src_v[pl.ds(i * 16, 16)]
            m = (d16 >= lo) & (d16 < lo + RPW)
            idx = (d16 - lo) * N + s16
            plsc.addupdate_scatter(slab_v, [idx], ones, mask=m)
            return carry

        lax.fori_loop(0, ESL // 16, ebody, 0, unroll=False)
        pltpu.sync_copy(slab_v, a_hbm.at[ep, pl.ds(lo * N, RPW * N)])

    return k(src, dst, zeros).reshape(EPART, N, N)


def _gat_body(z_ref, A_ref, emb_ref,
              wr_ref, wz_ref, wn_ref,
              bir_ref, biz_ref, bin_ref,
              bhr_ref, bhz_ref, bhn_ref,
              gatw_ref, atti_ref, attj_ref, gatb_ref,
              gnng_ref, gnnb_ref, bnog_ref, bnob_ref,
              outw_ref, outb_ref,
              out_ref,
              xp_s, emb_s, emb4_s, buf_s, st1_s, st2_s, A_s):
    i = pl.program_id(0)

    @pl.when(i == 0)
    def _init():
        es = []
        for p in range(NPROC):
            e = emb_ref[p]
            for l in range(3):
                hs = []
                for dr in range(2):
                    idx = (p * 3 + l) * 2 + dr
                    gr = jax.lax.dot_general(
                        e, wr_ref[idx], (((1,), (1,)), ((), ())),
                        preferred_element_type=jnp.float32) + bir_ref[idx:idx + 1]
                    gz = jax.lax.dot_general(
                        e, wz_ref[idx], (((1,), (1,)), ((), ())),
                        preferred_element_type=jnp.float32) + biz_ref[idx:idx + 1]
                    gn = jax.lax.dot_general(
                        e, wn_ref[idx], (((1,), (1,)), ((), ())),
                        preferred_element_type=jnp.float32) + bin_ref[idx:idx + 1]
                    r = jax.nn.sigmoid(gr + bhr_ref[idx:idx + 1])
                    zg = jax.nn.sigmoid(gz + bhz_ref[idx:idx + 1])
                    nn_ = jnp.tanh(gn + r * bhn_ref[idx:idx + 1])
                    hs.append((1.0 - zg) * nn_)
                e = jnp.concatenate(hs, axis=1)
            es.append(e)
        embfull = jnp.concatenate(es, axis=0)
        emb_s[...] = embfull
        emb4_s[...] = jnp.concatenate([embfull] * (CH // N), axis=0)
        st1_s[...] = jnp.zeros((2, DIM), jnp.float32)
        st2_s[...] = jnp.zeros((2, DIM), jnp.float32)
        A_s[...] = ((A_ref[0] + A_ref[1]) + (A_ref[2] + A_ref[3]))

    def _bclane(col, n):
        return jnp.broadcast_to(col, (col.shape[0], n))

    @pl.when(i < B)
    def _p0():
        b = i
        zb = z_ref[pl.ds(b * N, N), :]
        xpb = jnp.dot(zb, gatw_ref[...], preferred_element_type=jnp.float32)
        xp_s[pl.ds(b * N, N), :] = xpb
        cat = jnp.concatenate([xpb, emb_s[...]], axis=1)
        ti = jax.lax.dot_general(cat, atti_ref[...], (((1,), (1,)), ((), ())),
                                 preferred_element_type=jnp.float32)
        tj = jax.lax.dot_general(attj_ref[...], cat, (((1,), (1,)), ((), ())),
                                 preferred_element_type=jnp.float32)
        t = _bclane(ti, N) + tj
        alpha = jnp.where(t >= 0, t, 0.2 * t)
        A = A_s[...]
        mask = A > 0
        am = jnp.max(jnp.where(mask, alpha, -1e30), axis=1, keepdims=True)
        am = jnp.where(am > -1e29, am, 0.0)
        P = A * jnp.where(mask, jnp.exp(alpha - _bclane(am, N)), 0.0)
        den = jnp.sum(P, axis=1, keepdims=True)
        agg = jnp.dot(P, xpb, preferred_element_type=jnp.float32,
                      precision=jax.lax.Precision.HIGHEST)
        aggu = agg / (_bclane(den, DIM) + 1e-16) + gatb_ref[...]
        buf_s[pl.ds(b * N, N), :] = aggu
        st1_s[0:1, :] += jnp.sum(aggu, axis=0, keepdims=True)

    cnt = float(B * N)

    @pl.when((i >= B) & (i < B + NCH))
    def _p1v():
        c = i - B
        m = st1_s[0:1, :] / cnt
        dev = buf_s[pl.ds(c * CH, CH), :] - m
        st1_s[1:2, :] += jnp.sum(dev * dev, axis=0, keepdims=True)

    @pl.when((i >= B + NCH) & (i < B + 2 * NCH))
    def _p2():
        c = i - (B + NCH)
        m = st1_s[0:1, :] / cnt
        v = st1_s[1:2, :] / cnt
        af = buf_s[pl.ds(c * CH, CH), :]
        gcn = jnp.maximum(
            (af - m) / jnp.sqrt(v + 1e-5) * gnng_ref[...] + gnnb_ref[...], 0.0)
        of = gcn * emb4_s[...]
        buf_s[pl.ds(c * CH, CH), :] = of
        st2_s[0:1, :] += jnp.sum(of, axis=0, keepdims=True)

    @pl.when((i >= B + 2 * NCH) & (i < B + 3 * NCH))
    def _p3v():
        c = i - (B + 2 * NCH)
        m = st2_s[0:1, :] / cnt
        dev = buf_s[pl.ds(c * CH, CH), :] - m
        st2_s[1:2, :] += jnp.sum(dev * dev, axis=0, keepdims=True)

    @pl.when(i >= B + 3 * NCH)
    def _p4():
        m = st2_s[0:1, :] / cnt
        v = st2_s[1:2, :] / cnt
        c = i - (B + 3 * NCH)
        of = buf_s[pl.ds(c * CH, CH), :]
        o = jnp.maximum(
            (of - m) / jnp.sqrt(v + 1e-5) * bnog_ref[...] + bnob_ref[...], 0.0)
        res = jax.lax.dot_general(outw_ref[...], o, (((1,), (1,)), ((), ())),
                                  preferred_element_type=jnp.float32)
        out_ref[...] = res + outb_ref[0, 0]


def _gat(z, A, emb, wr, wz, wn, bir, biz, bin_, bhr, bhz, bhn,
         gatw, atti, attj, gatb, gnng, gnnb, bnog, bnob, outw, outb):
    full = lambda shape: pl.BlockSpec(shape, lambda i: tuple(0 for _ in shape))
    G = NPROC * 3 * 2
    return pl.pallas_call(
        _gat_body,
        grid=(NSTEP,),
        in_specs=[
            full((B * N, W)),
            full((EPART, N, N)),
            full((NPROC, DIM, DIM)),
            full((G, H, DIM)), full((G, H, DIM)), full((G, H, DIM)),
            full((G, H)), full((G, H)), full((G, H)),
            full((G, H)), full((G, H)), full((G, H)),
            full((W, DIM)),
            full((1, 2 * DIM)), full((1, 2 * DIM)),
            full((1, DIM)),
            full((1, DIM)), full((1, DIM)), full((1, DIM)), full((1, DIM)),
            full((1, DIM)), full((1, 1)),
        ],
        out_specs=pl.BlockSpec(
            (1, CH), lambda i: (0, jnp.maximum(i - (B + 3 * NCH), 0))),
        out_shape=jax.ShapeDtypeStruct((1, B * N), jnp.float32),
        scratch_shapes=[
            pltpu.VMEM((B * N, DIM), jnp.float32),
            pltpu.VMEM((N, DIM), jnp.float32),
            pltpu.VMEM((CH, DIM), jnp.float32),
            pltpu.VMEM((B * N, DIM), jnp.float32),
            pltpu.VMEM((2, DIM), jnp.float32),
            pltpu.VMEM((2, DIM), jnp.float32),
            pltpu.VMEM((N, N), jnp.float32),
        ],
    )(z, A, emb, wr, wz, wn, bir, biz, bin_, bhr, bhz, bhn,
      gatw, atti, attj, gatb, gnng, gnnb, bnog, bnob, outw, outb)


def kernel(data, target, org_edge_index, emb_tables, gru_Wih, gru_Whh,
           gru_bih, gru_bhh, enc_W, enc_b, enc_g, enc_beta, dec_W, dec_b,
           dec_g, dec_beta, gat_W, att_i, att_j, gat_b, gnn_g, gnn_beta,
           bno_g, bno_beta, out_W, out_b):
    x = data.reshape(B, D)
    z = _ae(x, enc_W, enc_b, enc_g, enc_beta)
    xr = _ae(z, dec_W, dec_b, dec_g, dec_beta)

    eidx = org_edge_index.astype(jnp.int32)
    A = _build_A(eidx[0], eidx[1], jnp.zeros((RPW * N,), jnp.float32))

    G = NPROC * 3 * 2
    wih = gru_Wih.reshape(G, 3 * H, DIM)
    wr, wz, wn = wih[:, :H, :], wih[:, H:2 * H, :], wih[:, 2 * H:, :]
    bih = gru_bih.reshape(G, 3 * H)
    bir, biz, bin_ = bih[:, :H], bih[:, H:2 * H], bih[:, 2 * H:]
    bhh = gru_bhh.reshape(G, 3 * H)
    bhr, bhz, bhn = bhh[:, :H], bhh[:, H:2 * H], bhh[:, 2 * H:]

    atti = att_i.reshape(1, 2 * DIM)
    attj = att_j.reshape(1, 2 * DIM)

    out = _gat(z.reshape(B * N, W), A, emb_tables,
               wr, wz, wn, bir, biz, bin_, bhr, bhz, bhn,
               gat_W, atti, attj, gat_b.reshape(1, DIM),
               gnn_g.reshape(1, DIM), gnn_beta.reshape(1, DIM),
               bno_g.reshape(1, DIM), bno_beta.reshape(1, DIM),
               out_W.reshape(1, DIM), out_b.reshape(1, 1))

    return (out.reshape(B, N), xr.reshape(B, N, W), z.reshape(B, N, W))


# speedup vs baseline: 1.1430x; 1.1430x over previous
import functools

import jax
import jax.numpy as jnp
from jax import lax
from jax.experimental import pallas as pl
from jax.experimental.pallas import tpu as pltpu
from jax.experimental.pallas import tpu_sc as plsc

B, N, W, DIM, H, NPROC = 32, 256, 16, 64, 32, 4
E_ORG = 16384
D = N * W
CBLK = 512
C = D // CBLK
CH = 1024
NCH = B * N // CH
BPS = 2
PB = B // BPS
NSTEP = PB + 4 * NCH


def _ae_body(x_ref, w_ref, b_ref, g_ref, bt_ref, o_ref, zmid):
    l = pl.program_id(0)
    c = pl.program_id(1)

    def layer(zin):
        h = jax.lax.dot_general(zin, w_ref[0], (((1,), (1,)), ((), ())),
                                preferred_element_type=jnp.float32)
        h = h + b_ref[0, 0]
        m = jnp.mean(h, axis=0, keepdims=True)
        v = jnp.mean((h - m) * (h - m), axis=0, keepdims=True)
        return jax.nn.sigmoid(
            (h - m) / jnp.sqrt(v + 1e-5) * g_ref[0, 0] + bt_ref[0, 0])

    @pl.when(l == 0)
    def _():
        zb = layer(x_ref[...])
        zmid[:, pl.ds(c * CBLK, CBLK)] = zb
        o_ref[...] = zb

    @pl.when(l == 1)
    def _():
        o_ref[...] = layer(zmid[...])


def _ae(x, Ws, bs, gs, bts):
    return pl.pallas_call(
        _ae_body,
        grid=(2, C),
        in_specs=[
            pl.BlockSpec((B, D), lambda l, c: (0, 0)),
            pl.BlockSpec((1, CBLK, D), lambda l, c: (l, c, 0)),
            pl.BlockSpec((1, 1, 1, CBLK), lambda l, c: (l, c, 0, 0)),
            pl.BlockSpec((1, 1, 1, CBLK), lambda l, c: (l, c, 0, 0)),
            pl.BlockSpec((1, 1, 1, CBLK), lambda l, c: (l, c, 0, 0)),
        ],
        out_specs=pl.BlockSpec((B, CBLK), lambda l, c: (0, c)),
        out_shape=jax.ShapeDtypeStruct((B, D), jnp.float32),
        scratch_shapes=[pltpu.VMEM((B, D), jnp.float32)],
    )(x, Ws, bs.reshape(2, C, 1, CBLK), gs.reshape(2, C, 1, CBLK),
      bts.reshape(2, C, 1, CBLK))


EPART = 4
RG = 8
RPW = N // RG
ESL = E_ORG // EPART


def _build_A(src, dst, zeros):
    info = plsc.get_sparse_core_info()
    mesh = plsc.VectorSubcoreMesh(core_axis_name="c", subcore_axis_name="s")

    @functools.partial(
        pl.kernel, mesh=mesh,
        compiler_params=pltpu.CompilerParams(needs_layout_passes=False),
        out_type=jax.ShapeDtypeStruct((EPART, N * N), jnp.float32),
        scratch_types=[
            pltpu.VMEM((ESL,), jnp.int32),
            pltpu.VMEM((ESL,), jnp.int32),
            pltpu.VMEM((RPW * N,), jnp.float32),
        ],
    )
    def k(src_hbm, dst_hbm, z_hbm, a_hbm, src_v, dst_v, slab_v):
        wid = lax.axis_index("s") * info.num_cores + lax.axis_index("c")
        g = wid // EPART
        ep = wid % EPART
        lo = g * RPW
        pltpu.sync_copy(src_hbm.at[pl.ds(ep * ESL, ESL)], src_v)
        pltpu.sync_copy(dst_hbm.at[pl.ds(ep * ESL, ESL)], dst_v)
        pltpu.sync_copy(z_hbm, slab_v)

        ones = jnp.ones((16,), jnp.float32)

        def ebody(i, carry):
            d16 = dst_v[pl.ds(i * 16, 16)]
            s16 = src_v[pl.ds(i * 16, 16)]
            m = (d16 >= lo) & (d16 < lo + RPW)
            idx = (d16 - lo) * N + s16
            plsc.addupdate_scatter(slab_v, [idx], ones, mask=m)
            return carry

        lax.fori_loop(0, ESL // 16, ebody, 0, unroll=False)
        pltpu.sync_copy(slab_v, a_hbm.at[ep, pl.ds(lo * N, RPW * N)])

    return k(src, dst, zeros).reshape(EPART, N, N)


def _gat_body(z_ref, A_ref, emb_ref,
              wr_ref, wz_ref, wn_ref,
              bir_ref, biz_ref, bin_ref,
              bhr_ref, bhz_ref, bhn_ref,
              gatw_ref, atti_ref, attj_ref, gatb_ref,
              gnng_ref, gnnb_ref, bnog_ref, bnob_ref,
              outw_ref, outb_ref,
              out_ref,
              xp_s, emb_s, emb4_s, buf_s, st1_s, st2_s, A_s):
    i = pl.program_id(0)

    @pl.when(i == 0)
    def _init():
        es = []
        for p in range(NPROC):
            e = emb_ref[p]
            for l in range(3):
                hs = []
                for dr in range(2):
                    idx = (p * 3 + l) * 2 + dr
                    gr = jax.lax.dot_general(
                        e, wr_ref[idx], (((1,), (1,)), ((), ())),
                        preferred_element_type=jnp.float32) + bir_ref[idx:idx + 1]
                    gz = jax.lax.dot_general(
                        e, wz_ref[idx], (((1,), (1,)), ((), ())),
                        preferred_element_type=jnp.float32) + biz_ref[idx:idx + 1]
                    gn = jax.lax.dot_general(
                        e, wn_ref[idx], (((1,), (1,)), ((), ())),
                        preferred_element_type=jnp.float32) + bin_ref[idx:idx + 1]
                    r = jax.nn.sigmoid(gr + bhr_ref[idx:idx + 1])
                    zg = jax.nn.sigmoid(gz + bhz_ref[idx:idx + 1])
                    nn_ = jnp.tanh(gn + r * bhn_ref[idx:idx + 1])
                    hs.append((1.0 - zg) * nn_)
                e = jnp.concatenate(hs, axis=1)
            es.append(e)
        embfull = jnp.concatenate(es, axis=0)
        emb_s[...] = embfull
        emb4_s[...] = jnp.concatenate([embfull] * (CH // N), axis=0)
        st1_s[...] = jnp.zeros((2, DIM), jnp.float32)
        st2_s[...] = jnp.zeros((2, DIM), jnp.float32)
        A_s[...] = ((A_ref[0] + A_ref[1]) + (A_ref[2] + A_ref[3]))

    def _bclane(col, n):
        return jnp.broadcast_to(col, (col.shape[0], n))

    @pl.when(i < PB)
    def _p0():
        for k in range(BPS):
            b = i * BPS + k
            zb = z_ref[pl.ds(b * N, N), :]
            xpb = jnp.dot(zb, gatw_ref[...], preferred_element_type=jnp.float32)
            xp_s[pl.ds(b * N, N), :] = xpb
            cat = jnp.concatenate([xpb, emb_s[...]], axis=1)
            ti = jax.lax.dot_general(cat, atti_ref[...], (((1,), (1,)), ((), ())),
                                     preferred_element_type=jnp.float32)
            tj = jax.lax.dot_general(attj_ref[...], cat, (((1,), (1,)), ((), ())),
                                     preferred_element_type=jnp.float32)
            t = _bclane(ti, N) + tj
            alpha = jnp.where(t >= 0, t, 0.2 * t)
            A = A_s[...]
            mask = A > 0
            am = jnp.max(jnp.where(mask, alpha, -1e30), axis=1, keepdims=True)
            am = jnp.where(am > -1e29, am, 0.0)
            P = A * jnp.where(mask, jnp.exp(alpha - _bclane(am, N)), 0.0)
            den = jnp.sum(P, axis=1, keepdims=True)
            agg = jnp.dot(P, xpb, preferred_element_type=jnp.float32,
                          precision=jax.lax.Precision.HIGHEST)
            aggu = agg / (_bclane(den, DIM) + 1e-16) + gatb_ref[...]
            buf_s[pl.ds(b * N, N), :] = aggu
            st1_s[0:1, :] += jnp.sum(aggu, axis=0, keepdims=True)

    cnt = float(B * N)

    @pl.when((i >= PB) & (i < PB + NCH))
    def _p1v():
        c = i - PB
        m = st1_s[0:1, :] / cnt
        dev = buf_s[pl.ds(c * CH, CH), :] - m
        st1_s[1:2, :] += jnp.sum(dev * dev, axis=0, keepdims=True)

    @pl.when((i >= PB + NCH) & (i < PB + 2 * NCH))
    def _p2():
        c = i - (PB + NCH)
        m = st1_s[0:1, :] / cnt
        v = st1_s[1:2, :] / cnt
        af = buf_s[pl.ds(c * CH, CH), :]
        gcn = jnp.maximum(
            (af - m) / jnp.sqrt(v + 1e-5) * gnng_ref[...] + gnnb_ref[...], 0.0)
        of = gcn * emb4_s[...]
        buf_s[pl.ds(c * CH, CH), :] = of
        st2_s[0:1, :] += jnp.sum(of, axis=0, keepdims=True)

    @pl.when((i >= PB + 2 * NCH) & (i < PB + 3 * NCH))
    def _p3v():
        c = i - (PB + 2 * NCH)
        m = st2_s[0:1, :] / cnt
        dev = buf_s[pl.ds(c * CH, CH), :] - m
        st2_s[1:2, :] += jnp.sum(dev * dev, axis=0, keepdims=True)

    @pl.when(i >= PB + 3 * NCH)
    def _p4():
        m = st2_s[0:1, :] / cnt
        v = st2_s[1:2, :] / cnt
        c = i - (PB + 3 * NCH)
        of = buf_s[pl.ds(c * CH, CH), :]
        o = jnp.maximum(
            (of - m) / jnp.sqrt(v + 1e-5) * bnog_ref[...] + bnob_ref[...], 0.0)
        res = jax.lax.dot_general(outw_ref[...], o, (((1,), (1,)), ((), ())),
                                  preferred_element_type=jnp.float32)
        out_ref[...] = res + outb_ref[0, 0]


def _gat(z, A, emb, wr, wz, wn, bir, biz, bin_, bhr, bhz, bhn,
         gatw, atti, attj, gatb, gnng, gnnb, bnog, bnob, outw, outb):
    full = lambda shape: pl.BlockSpec(shape, lambda i: tuple(0 for _ in shape))
    G = NPROC * 3 * 2
    return pl.pallas_call(
        _gat_body,
        grid=(NSTEP,),
        in_specs=[
            full((B * N, W)),
            full((EPART, N, N)),
            full((NPROC, DIM, DIM)),
            full((G, H, DIM)), full((G, H, DIM)), full((G, H, DIM)),
            full((G, H)), full((G, H)), full((G, H)),
            full((G, H)), full((G, H)), full((G, H)),
            full((W, DIM)),
            full((1, 2 * DIM)), full((1, 2 * DIM)),
            full((1, DIM)),
            full((1, DIM)), full((1, DIM)), full((1, DIM)), full((1, DIM)),
            full((1, DIM)), full((1, 1)),
        ],
        out_specs=pl.BlockSpec(
            (1, CH), lambda i: (0, jnp.maximum(i - (PB + 3 * NCH), 0))),
        out_shape=jax.ShapeDtypeStruct((1, B * N), jnp.float32),
        scratch_shapes=[
            pltpu.VMEM((B * N, DIM), jnp.float32),
            pltpu.VMEM((N, DIM), jnp.float32),
            pltpu.VMEM((CH, DIM), jnp.float32),
            pltpu.VMEM((B * N, DIM), jnp.float32),
            pltpu.VMEM((2, DIM), jnp.float32),
            pltpu.VMEM((2, DIM), jnp.float32),
            pltpu.VMEM((N, N), jnp.float32),
        ],
    )(z, A, emb, wr, wz, wn, bir, biz, bin_, bhr, bhz, bhn,
      gatw, atti, attj, gatb, gnng, gnnb, bnog, bnob, outw, outb)


def kernel(data, target, org_edge_index, emb_tables, gru_Wih, gru_Whh,
           gru_bih, gru_bhh, enc_W, enc_b, enc_g, enc_beta, dec_W, dec_b,
           dec_g, dec_beta, gat_W, att_i, att_j, gat_b, gnn_g, gnn_beta,
           bno_g, bno_beta, out_W, out_b):
    x = data.reshape(B, D)
    z = _ae(x, enc_W, enc_b, enc_g, enc_beta)
    xr = _ae(z, dec_W, dec_b, dec_g, dec_beta)

    eidx = org_edge_index.astype(jnp.int32)
    A = _build_A(eidx[0], eidx[1], jnp.zeros((RPW * N,), jnp.float32))

    G = NPROC * 3 * 2
    wih = gru_Wih.reshape(G, 3 * H, DIM)
    wr, wz, wn = wih[:, :H, :], wih[:, H:2 * H, :], wih[:, 2 * H:, :]
    bih = gru_bih.reshape(G, 3 * H)
    bir, biz, bin_ = bih[:, :H], bih[:, H:2 * H], bih[:, 2 * H:]
    bhh = gru_bhh.reshape(G, 3 * H)
    bhr, bhz, bhn = bhh[:, :H], bhh[:, H:2 * H], bhh[:, 2 * H:]

    atti = att_i.reshape(1, 2 * DIM)
    attj = att_j.reshape(1, 2 * DIM)

    out = _gat(z.reshape(B * N, W), A, emb_tables,
               wr, wz, wn, bir, biz, bin_, bhr, bhz, bhn,
               gat_W, atti, attj, gat_b.reshape(1, DIM),
               gnn_g.reshape(1, DIM), gnn_beta.reshape(1, DIM),
               bno_g.reshape(1, DIM), bno_beta.reshape(1, DIM),
               out_W.reshape(1, DIM), out_b.reshape(1, 1))

    return (out.reshape(B, N), xr.reshape(B, N, W), z.reshape(B, N, W))


# speedup vs baseline: 1.1758x; 1.0286x over previous
import functools

import jax
import jax.numpy as jnp
from jax import lax
from jax.experimental import pallas as pl
from jax.experimental.pallas import tpu as pltpu
from jax.experimental.pallas import tpu_sc as plsc

B, N, W, DIM, H, NPROC = 32, 256, 16, 64, 32, 4
E_ORG = 16384
D = N * W
CBLK = 512
C = D // CBLK
CH = 2048
NCH = B * N // CH
BPS = 4
PB = B // BPS
NSTEP = PB + 4 * NCH


def _ae_body(x_ref, w_ref, b_ref, g_ref, bt_ref, o_ref, zmid):
    l = pl.program_id(0)
    c = pl.program_id(1)

    def layer(zin):
        h = jax.lax.dot_general(zin, w_ref[0], (((1,), (1,)), ((), ())),
                                preferred_element_type=jnp.float32)
        h = h + b_ref[0, 0]
        m = jnp.mean(h, axis=0, keepdims=True)
        v = jnp.mean((h - m) * (h - m), axis=0, keepdims=True)
        return jax.nn.sigmoid(
            (h - m) / jnp.sqrt(v + 1e-5) * g_ref[0, 0] + bt_ref[0, 0])

    @pl.when(l == 0)
    def _():
        zb = layer(x_ref[...])
        zmid[:, pl.ds(c * CBLK, CBLK)] = zb
        o_ref[...] = zb

    @pl.when(l == 1)
    def _():
        o_ref[...] = layer(zmid[...])


def _ae(x, Ws, bs, gs, bts):
    return pl.pallas_call(
        _ae_body,
        grid=(2, C),
        in_specs=[
            pl.BlockSpec((B, D), lambda l, c: (0, 0)),
            pl.BlockSpec((1, CBLK, D), lambda l, c: (l, c, 0)),
            pl.BlockSpec((1, 1, 1, CBLK), lambda l, c: (l, c, 0, 0)),
            pl.BlockSpec((1, 1, 1, CBLK), lambda l, c: (l, c, 0, 0)),
            pl.BlockSpec((1, 1, 1, CBLK), lambda l, c: (l, c, 0, 0)),
        ],
        out_specs=pl.BlockSpec((B, CBLK), lambda l, c: (0, c)),
        out_shape=jax.ShapeDtypeStruct((B, D), jnp.float32),
        scratch_shapes=[pltpu.VMEM((B, D), jnp.float32)],
    )(x, Ws, bs.reshape(2, C, 1, CBLK), gs.reshape(2, C, 1, CBLK),
      bts.reshape(2, C, 1, CBLK))


EPART = 4
RG = 8
RPW = N // RG
ESL = E_ORG // EPART


def _build_A(src, dst, zeros):
    info = plsc.get_sparse_core_info()
    mesh = plsc.VectorSubcoreMesh(core_axis_name="c", subcore_axis_name="s")

    @functools.partial(
        pl.kernel, mesh=mesh,
        compiler_params=pltpu.CompilerParams(needs_layout_passes=False),
        out_type=jax.ShapeDtypeStruct((EPART, N * N), jnp.float32),
        scratch_types=[
            pltpu.VMEM((ESL,), jnp.int32),
            pltpu.VMEM((ESL,), jnp.int32),
            pltpu.VMEM((RPW * N,), jnp.float32),
        ],
    )
    def k(src_hbm, dst_hbm, z_hbm, a_hbm, src_v, dst_v, slab_v):
        wid = lax.axis_index("s") * info.num_cores + lax.axis_index("c")
        g = wid // EPART
        ep = wid % EPART
        lo = g * RPW
        pltpu.sync_copy(src_hbm.at[pl.ds(ep * ESL, ESL)], src_v)
        pltpu.sync_copy(dst_hbm.at[pl.ds(ep * ESL, ESL)], dst_v)
        pltpu.sync_copy(z_hbm, slab_v)

        ones = jnp.ones((16,), jnp.float32)

        def ebody(i, carry):
            d16 = dst_v[pl.ds(i * 16, 16)]
            s16 = src_v[pl.ds(i * 16, 16)]
            m = (d16 >= lo) & (d16 < lo + RPW)
            idx = (d16 - lo) * N + s16
            plsc.addupdate_scatter(slab_v, [idx], ones, mask=m)
            return carry

        lax.fori_loop(0, ESL // 16, ebody, 0, unroll=False)
        pltpu.sync_copy(slab_v, a_hbm.at[ep, pl.ds(lo * N, RPW * N)])

    return k(src, dst, zeros).reshape(EPART, N, N)


def _gat_body(z_ref, A_ref, emb_ref,
              wr_ref, wz_ref, wn_ref,
              bir_ref, biz_ref, bin_ref,
              bhr_ref, bhz_ref, bhn_ref,
              gatw_ref, atti_ref, attj_ref, gatb_ref,
              gnng_ref, gnnb_ref, bnog_ref, bnob_ref,
              outw_ref, outb_ref,
              out_ref,
              xp_s, emb_s, emb4_s, buf_s, st1_s, st2_s, A_s):
    i = pl.program_id(0)

    @pl.when(i == 0)
    def _init():
        es = []
        for p in range(NPROC):
            e = emb_ref[p]
            for l in range(3):
                hs = []
                for dr in range(2):
                    idx = (p * 3 + l) * 2 + dr
                    gr = jax.lax.dot_general(
                        e, wr_ref[idx], (((1,), (1,)), ((), ())),
                        preferred_element_type=jnp.float32) + bir_ref[idx:idx + 1]
                    gz = jax.lax.dot_general(
                        e, wz_ref[idx], (((1,), (1,)), ((), ())),
                        preferred_element_type=jnp.float32) + biz_ref[idx:idx + 1]
                    gn = jax.lax.dot_general(
                        e, wn_ref[idx], (((1,), (1,)), ((), ())),
                        preferred_element_type=jnp.float32) + bin_ref[idx:idx + 1]
                    r = jax.nn.sigmoid(gr + bhr_ref[idx:idx + 1])
                    zg = jax.nn.sigmoid(gz + bhz_ref[idx:idx + 1])
                    nn_ = jnp.tanh(gn + r * bhn_ref[idx:idx + 1])
                    hs.append((1.0 - zg) * nn_)
                e = jnp.concatenate(hs, axis=1)
            es.append(e)
        embfull = jnp.concatenate(es, axis=0)
        emb_s[...] = embfull
        emb4_s[...] = jnp.concatenate([embfull] * (CH // N), axis=0)
        st1_s[...] = jnp.zeros((2, DIM), jnp.float32)
        st2_s[...] = jnp.zeros((2, DIM), jnp.float32)
        A_s[...] = ((A_ref[0] + A_ref[1]) + (A_ref[2] + A_ref[3]))

    def _bclane(col, n):
        return jnp.broadcast_to(col, (col.shape[0], n))

    @pl.when(i < PB)
    def _p0():
        for k in range(BPS):
            b = i * BPS + k
            zb = z_ref[pl.ds(b * N, N), :]
            xpb = jnp.dot(zb, gatw_ref[...], preferred_element_type=jnp.float32)
            xp_s[pl.ds(b * N, N), :] = xpb
            cat = jnp.concatenate([xpb, emb_s[...]], axis=1)
            ti = jax.lax.dot_general(cat, atti_ref[...], (((1,), (1,)), ((), ())),
                                     preferred_element_type=jnp.float32)
            tj = jax.lax.dot_general(attj_ref[...], cat, (((1,), (1,)), ((), ())),
                                     preferred_element_type=jnp.float32)
            t = _bclane(ti, N) + tj
            alpha = jnp.where(t >= 0, t, 0.2 * t)
            A = A_s[...]
            mask = A > 0
            am = jnp.max(jnp.where(mask, alpha, -1e30), axis=1, keepdims=True)
            am = jnp.where(am > -1e29, am, 0.0)
            P = A * jnp.where(mask, jnp.exp(alpha - _bclane(am, N)), 0.0)
            den = jnp.sum(P, axis=1, keepdims=True)
            agg = jnp.dot(P, xpb, preferred_element_type=jnp.float32,
                          precision=jax.lax.Precision.HIGHEST)
            aggu = agg / (_bclane(den, DIM) + 1e-16) + gatb_ref[...]
            buf_s[pl.ds(b * N, N), :] = aggu
            st1_s[0:1, :] += jnp.sum(aggu, axis=0, keepdims=True)

    cnt = float(B * N)

    @pl.when((i >= PB) & (i < PB + NCH))
    def _p1v():
        c = i - PB
        m = st1_s[0:1, :] / cnt
        dev = buf_s[pl.ds(c * CH, CH), :] - m
        st1_s[1:2, :] += jnp.sum(dev * dev, axis=0, keepdims=True)

    @pl.when((i >= PB + NCH) & (i < PB + 2 * NCH))
    def _p2():
        c = i - (PB + NCH)
        m = st1_s[0:1, :] / cnt
        v = st1_s[1:2, :] / cnt
        af = buf_s[pl.ds(c * CH, CH), :]
        gcn = jnp.maximum(
            (af - m) / jnp.sqrt(v + 1e-5) * gnng_ref[...] + gnnb_ref[...], 0.0)
        of = gcn * emb4_s[...]
        buf_s[pl.ds(c * CH, CH), :] = of
        st2_s[0:1, :] += jnp.sum(of, axis=0, keepdims=True)

    @pl.when((i >= PB + 2 * NCH) & (i < PB + 3 * NCH))
    def _p3v():
        c = i - (PB + 2 * NCH)
        m = st2_s[0:1, :] / cnt
        dev = buf_s[pl.ds(c * CH, CH), :] - m
        st2_s[1:2, :] += jnp.sum(dev * dev, axis=0, keepdims=True)

    @pl.when(i >= PB + 3 * NCH)
    def _p4():
        m = st2_s[0:1, :] / cnt
        v = st2_s[1:2, :] / cnt
        c = i - (PB + 3 * NCH)
        of = buf_s[pl.ds(c * CH, CH), :]
        o = jnp.maximum(
            (of - m) / jnp.sqrt(v + 1e-5) * bnog_ref[...] + bnob_ref[...], 0.0)
        res = jax.lax.dot_general(outw_ref[...], o, (((1,), (1,)), ((), ())),
                                  preferred_element_type=jnp.float32)
        out_ref[...] = res + outb_ref[0, 0]


def _gat(z, A, emb, wr, wz, wn, bir, biz, bin_, bhr, bhz, bhn,
         gatw, atti, attj, gatb, gnng, gnnb, bnog, bnob, outw, outb):
    full = lambda shape: pl.BlockSpec(shape, lambda i: tuple(0 for _ in shape))
    G = NPROC * 3 * 2
    return pl.pallas_call(
        _gat_body,
        grid=(NSTEP,),
        in_specs=[
            full((B * N, W)),
            full((EPART, N, N)),
            full((NPROC, DIM, DIM)),
            full((G, H, DIM)), full((G, H, DIM)), full((G, H, DIM)),
            full((G, H)), full((G, H)), full((G, H)),
            full((G, H)), full((G, H)), full((G, H)),
            full((W, DIM)),
            full((1, 2 * DIM)), full((1, 2 * DIM)),
            full((1, DIM)),
            full((1, DIM)), full((1, DIM)), full((1, DIM)), full((1, DIM)),
            full((1, DIM)), full((1, 1)),
        ],
        out_specs=pl.BlockSpec(
            (1, CH), lambda i: (0, jnp.maximum(i - (PB + 3 * NCH), 0))),
        out_shape=jax.ShapeDtypeStruct((1, B * N), jnp.float32),
        scratch_shapes=[
            pltpu.VMEM((B * N, DIM), jnp.float32),
            pltpu.VMEM((N, DIM), jnp.float32),
            pltpu.VMEM((CH, DIM), jnp.float32),
            pltpu.VMEM((B * N, DIM), jnp.float32),
            pltpu.VMEM((2, DIM), jnp.float32),
            pltpu.VMEM((2, DIM), jnp.float32),
            pltpu.VMEM((N, N), jnp.float32),
        ],
    )(z, A, emb, wr, wz, wn, bir, biz, bin_, bhr, bhz, bhn,
      gatw, atti, attj, gatb, gnng, gnnb, bnog, bnob, outw, outb)


def kernel(data, target, org_edge_index, emb_tables, gru_Wih, gru_Whh,
           gru_bih, gru_bhh, enc_W, enc_b, enc_g, enc_beta, dec_W, dec_b,
           dec_g, dec_beta, gat_W, att_i, att_j, gat_b, gnn_g, gnn_beta,
           bno_g, bno_beta, out_W, out_b):
    x = data.reshape(B, D)
    z = _ae(x, enc_W, enc_b, enc_g, enc_beta)
    xr = _ae(z, dec_W, dec_b, dec_g, dec_beta)

    eidx = org_edge_index.astype(jnp.int32)
    A = _build_A(eidx[0], eidx[1], jnp.zeros((RPW * N,), jnp.float32))

    G = NPROC * 3 * 2
    wih = gru_Wih.reshape(G, 3 * H, DIM)
    wr, wz, wn = wih[:, :H, :], wih[:, H:2 * H, :], wih[:, 2 * H:, :]
    bih = gru_bih.reshape(G, 3 * H)
    bir, biz, bin_ = bih[:, :H], bih[:, H:2 * H], bih[:, 2 * H:]
    bhh = gru_bhh.reshape(G, 3 * H)
    bhr, bhz, bhn = bhh[:, :H], bhh[:, H:2 * H], bhh[:, 2 * H:]

    atti = att_i.reshape(1, 2 * DIM)
    attj = att_j.reshape(1, 2 * DIM)

    out = _gat(z.reshape(B * N, W), A, emb_tables,
               wr, wz, wn, bir, biz, bin_, bhr, bhz, bhn,
               gat_W, atti, attj, gat_b.reshape(1, DIM),
               gnn_g.reshape(1, DIM), gnn_beta.reshape(1, DIM),
               bno_g.reshape(1, DIM), bno_beta.reshape(1, DIM),
               out_W.reshape(1, DIM), out_b.reshape(1, 1))

    return (out.reshape(B, N), xr.reshape(B, N, W), z.reshape(B, N, W))


# speedup vs baseline: 1.1832x; 1.0064x over previous
import functools

import jax
import jax.numpy as jnp
from jax import lax
from jax.experimental import pallas as pl
from jax.experimental.pallas import tpu as pltpu
from jax.experimental.pallas import tpu_sc as plsc

B, N, W, DIM, H, NPROC = 32, 256, 16, 64, 32, 4
E_ORG = 16384
D = N * W
CBLK = 512
C = D // CBLK
CH = 2048
NCH = B * N // CH
BPS = 8
PB = B // BPS
NSTEP = PB + 4 * NCH


def _ae_body(x_ref, w_ref, b_ref, g_ref, bt_ref, o_ref, zmid):
    l = pl.program_id(0)
    c = pl.program_id(1)

    def layer(zin):
        h = jax.lax.dot_general(zin, w_ref[0], (((1,), (1,)), ((), ())),
                                preferred_element_type=jnp.float32)
        h = h + b_ref[0, 0]
        m = jnp.mean(h, axis=0, keepdims=True)
        v = jnp.mean((h - m) * (h - m), axis=0, keepdims=True)
        return jax.nn.sigmoid(
            (h - m) / jnp.sqrt(v + 1e-5) * g_ref[0, 0] + bt_ref[0, 0])

    @pl.when(l == 0)
    def _():
        zb = layer(x_ref[...])
        zmid[:, pl.ds(c * CBLK, CBLK)] = zb
        o_ref[...] = zb

    @pl.when(l == 1)
    def _():
        o_ref[...] = layer(zmid[...])


def _ae(x, Ws, bs, gs, bts):
    return pl.pallas_call(
        _ae_body,
        grid=(2, C),
        in_specs=[
            pl.BlockSpec((B, D), lambda l, c: (0, 0)),
            pl.BlockSpec((1, CBLK, D), lambda l, c: (l, c, 0)),
            pl.BlockSpec((1, 1, 1, CBLK), lambda l, c: (l, c, 0, 0)),
            pl.BlockSpec((1, 1, 1, CBLK), lambda l, c: (l, c, 0, 0)),
            pl.BlockSpec((1, 1, 1, CBLK), lambda l, c: (l, c, 0, 0)),
        ],
        out_specs=pl.BlockSpec((B, CBLK), lambda l, c: (0, c)),
        out_shape=jax.ShapeDtypeStruct((B, D), jnp.float32),
        scratch_shapes=[pltpu.VMEM((B, D), jnp.float32)],
    )(x, Ws, bs.reshape(2, C, 1, CBLK), gs.reshape(2, C, 1, CBLK),
      bts.reshape(2, C, 1, CBLK))


EPART = 4
RG = 8
RPW = N // RG
ESL = E_ORG // EPART


def _build_A(src, dst, zeros):
    info = plsc.get_sparse_core_info()
    mesh = plsc.VectorSubcoreMesh(core_axis_name="c", subcore_axis_name="s")

    @functools.partial(
        pl.kernel, mesh=mesh,
        compiler_params=pltpu.CompilerParams(needs_layout_passes=False),
        out_type=jax.ShapeDtypeStruct((EPART, N * N), jnp.float32),
        scratch_types=[
            pltpu.VMEM((ESL,), jnp.int32),
            pltpu.VMEM((ESL,), jnp.int32),
            pltpu.VMEM((RPW * N,), jnp.float32),
        ],
    )
    def k(src_hbm, dst_hbm, z_hbm, a_hbm, src_v, dst_v, slab_v):
        wid = lax.axis_index("s") * info.num_cores + lax.axis_index("c")
        g = wid // EPART
        ep = wid % EPART
        lo = g * RPW
        pltpu.sync_copy(src_hbm.at[pl.ds(ep * ESL, ESL)], src_v)
        pltpu.sync_copy(dst_hbm.at[pl.ds(ep * ESL, ESL)], dst_v)
        pltpu.sync_copy(z_hbm, slab_v)

        ones = jnp.ones((16,), jnp.float32)

        def ebody(i, carry):
            d16 = dst_v[pl.ds(i * 16, 16)]
            s16 = src_v[pl.ds(i * 16, 16)]
            m = (d16 >= lo) & (d16 < lo + RPW)
            idx = (d16 - lo) * N + s16
            plsc.addupdate_scatter(slab_v, [idx], ones, mask=m)
            return carry

        lax.fori_loop(0, ESL // 16, ebody, 0, unroll=False)
        pltpu.sync_copy(slab_v, a_hbm.at[ep, pl.ds(lo * N, RPW * N)])

    return k(src, dst, zeros).reshape(EPART, N, N)


def _gat_body(z_ref, A_ref, emb_ref,
              wr_ref, wz_ref, wn_ref,
              bir_ref, biz_ref, bin_ref,
              bhr_ref, bhz_ref, bhn_ref,
              gatw_ref, atti_ref, attj_ref, gatb_ref,
              gnng_ref, gnnb_ref, bnog_ref, bnob_ref,
              outw_ref, outb_ref,
              out_ref,
              xp_s, emb_s, emb4_s, buf_s, st1_s, st2_s, A_s):
    i = pl.program_id(0)

    @pl.when(i == 0)
    def _init():
        es = []
        for p in range(NPROC):
            e = emb_ref[p]
            for l in range(3):
                hs = []
                for dr in range(2):
                    idx = (p * 3 + l) * 2 + dr
                    gr = jax.lax.dot_general(
                        e, wr_ref[idx], (((1,), (1,)), ((), ())),
                        preferred_element_type=jnp.float32) + bir_ref[idx:idx + 1]
                    gz = jax.lax.dot_general(
                        e, wz_ref[idx], (((1,), (1,)), ((), ())),
                        preferred_element_type=jnp.float32) + biz_ref[idx:idx + 1]
                    gn = jax.lax.dot_general(
                        e, wn_ref[idx], (((1,), (1,)), ((), ())),
                        preferred_element_type=jnp.float32) + bin_ref[idx:idx + 1]
                    r = jax.nn.sigmoid(gr + bhr_ref[idx:idx + 1])
                    zg = jax.nn.sigmoid(gz + bhz_ref[idx:idx + 1])
                    nn_ = jnp.tanh(gn + r * bhn_ref[idx:idx + 1])
                    hs.append((1.0 - zg) * nn_)
                e = jnp.concatenate(hs, axis=1)
            es.append(e)
        embfull = jnp.concatenate(es, axis=0)
        emb_s[...] = embfull
        emb4_s[...] = jnp.concatenate([embfull] * (CH // N), axis=0)
        st1_s[...] = jnp.zeros((2, DIM), jnp.float32)
        st2_s[...] = jnp.zeros((2, DIM), jnp.float32)
        A_s[...] = ((A_ref[0] + A_ref[1]) + (A_ref[2] + A_ref[3]))

    def _bclane(col, n):
        return jnp.broadcast_to(col, (col.shape[0], n))

    @pl.when(i < PB)
    def _p0():
        for k in range(BPS):
            b = i * BPS + k
            zb = z_ref[pl.ds(b * N, N), :]
            xpb = jnp.dot(zb, gatw_ref[...], preferred_element_type=jnp.float32)
            xp_s[pl.ds(b * N, N), :] = xpb
            cat = jnp.concatenate([xpb, emb_s[...]], axis=1)
            ti = jax.lax.dot_general(cat, atti_ref[...], (((1,), (1,)), ((), ())),
                                     preferred_element_type=jnp.float32)
            tj = jax.lax.dot_general(attj_ref[...], cat, (((1,), (1,)), ((), ())),
                                     preferred_element_type=jnp.float32)
            t = _bclane(ti, N) + tj
            alpha = jnp.where(t >= 0, t, 0.2 * t)
            A = A_s[...]
            mask = A > 0
            am = jnp.max(jnp.where(mask, alpha, -1e30), axis=1, keepdims=True)
            am = jnp.where(am > -1e29, am, 0.0)
            P = A * jnp.where(mask, jnp.exp(alpha - _bclane(am, N)), 0.0)
            den = jnp.sum(P, axis=1, keepdims=True)
            agg = jnp.dot(P, xpb, preferred_element_type=jnp.float32,
                          precision=jax.lax.Precision.HIGHEST)
            aggu = agg / (_bclane(den, DIM) + 1e-16) + gatb_ref[...]
            buf_s[pl.ds(b * N, N), :] = aggu
            st1_s[0:1, :] += jnp.sum(aggu, axis=0, keepdims=True)

    cnt = float(B * N)

    @pl.when((i >= PB) & (i < PB + NCH))
    def _p1v():
        c = i - PB
        m = st1_s[0:1, :] / cnt
        dev = buf_s[pl.ds(c * CH, CH), :] - m
        st1_s[1:2, :] += jnp.sum(dev * dev, axis=0, keepdims=True)

    @pl.when((i >= PB + NCH) & (i < PB + 2 * NCH))
    def _p2():
        c = i - (PB + NCH)
        m = st1_s[0:1, :] / cnt
        v = st1_s[1:2, :] / cnt
        af = buf_s[pl.ds(c * CH, CH), :]
        gcn = jnp.maximum(
            (af - m) / jnp.sqrt(v + 1e-5) * gnng_ref[...] + gnnb_ref[...], 0.0)
        of = gcn * emb4_s[...]
        buf_s[pl.ds(c * CH, CH), :] = of
        st2_s[0:1, :] += jnp.sum(of, axis=0, keepdims=True)

    @pl.when((i >= PB + 2 * NCH) & (i < PB + 3 * NCH))
    def _p3v():
        c = i - (PB + 2 * NCH)
        m = st2_s[0:1, :] / cnt
        dev = buf_s[pl.ds(c * CH, CH), :] - m
        st2_s[1:2, :] += jnp.sum(dev * dev, axis=0, keepdims=True)

    @pl.when(i >= PB + 3 * NCH)
    def _p4():
        m = st2_s[0:1, :] / cnt
        v = st2_s[1:2, :] / cnt
        c = i - (PB + 3 * NCH)
        of = buf_s[pl.ds(c * CH, CH), :]
        o = jnp.maximum(
            (of - m) / jnp.sqrt(v + 1e-5) * bnog_ref[...] + bnob_ref[...], 0.0)
        res = jax.lax.dot_general(outw_ref[...], o, (((1,), (1,)), ((), ())),
                                  preferred_element_type=jnp.float32)
        out_ref[...] = res + outb_ref[0, 0]


def _gat(z, A, emb, wr, wz, wn, bir, biz, bin_, bhr, bhz, bhn,
         gatw, atti, attj, gatb, gnng, gnnb, bnog, bnob, outw, outb):
    full = lambda shape: pl.BlockSpec(shape, lambda i: tuple(0 for _ in shape))
    G = NPROC * 3 * 2
    return pl.pallas_call(
        _gat_body,
        grid=(NSTEP,),
        in_specs=[
            full((B * N, W)),
            full((EPART, N, N)),
            full((NPROC, DIM, DIM)),
            full((G, H, DIM)), full((G, H, DIM)), full((G, H, DIM)),
            full((G, H)), full((G, H)), full((G, H)),
            full((G, H)), full((G, H)), full((G, H)),
            full((W, DIM)),
            full((1, 2 * DIM)), full((1, 2 * DIM)),
            full((1, DIM)),
            full((1, DIM)), full((1, DIM)), full((1, DIM)), full((1, DIM)),
            full((1, DIM)), full((1, 1)),
        ],
        out_specs=pl.BlockSpec(
            (1, CH), lambda i: (0, jnp.maximum(i - (PB + 3 * NCH), 0))),
        out_shape=jax.ShapeDtypeStruct((1, B * N), jnp.float32),
        scratch_shapes=[
            pltpu.VMEM((B * N, DIM), jnp.float32),
            pltpu.VMEM((N, DIM), jnp.float32),
            pltpu.VMEM((CH, DIM), jnp.float32),
            pltpu.VMEM((B * N, DIM), jnp.float32),
            pltpu.VMEM((2, DIM), jnp.float32),
            pltpu.VMEM((2, DIM), jnp.float32),
            pltpu.VMEM((N, N), jnp.float32),
        ],
    )(z, A, emb, wr, wz, wn, bir, biz, bin_, bhr, bhz, bhn,
      gatw, atti, attj, gatb, gnng, gnnb, bnog, bnob, outw, outb)


def kernel(data, target, org_edge_index, emb_tables, gru_Wih, gru_Whh,
           gru_bih, gru_bhh, enc_W, enc_b, enc_g, enc_beta, dec_W, dec_b,
           dec_g, dec_beta, gat_W, att_i, att_j, gat_b, gnn_g, gnn_beta,
           bno_g, bno_beta, out_W, out_b):
    x = data.reshape(B, D)
    z = _ae(x, enc_W, enc_b, enc_g, enc_beta)
    xr = _ae(z, dec_W, dec_b, dec_g, dec_beta)

    eidx = org_edge_index.astype(jnp.int32)
    A = _build_A(eidx[0], eidx[1], jnp.zeros((RPW * N,), jnp.float32))

    G = NPROC * 3 * 2
    wih = gru_Wih.reshape(G, 3 * H, DIM)
    wr, wz, wn = wih[:, :H, :], wih[:, H:2 * H, :], wih[:, 2 * H:, :]
    bih = gru_bih.reshape(G, 3 * H)
    bir, biz, bin_ = bih[:, :H], bih[:, H:2 * H], bih[:, 2 * H:]
    bhh = gru_bhh.reshape(G, 3 * H)
    bhr, bhz, bhn = bhh[:, :H], bhh[:, H:2 * H], bhh[:, 2 * H:]

    atti = att_i.reshape(1, 2 * DIM)
    attj = att_j.reshape(1, 2 * DIM)

    out = _gat(z.reshape(B * N, W), A, emb_tables,
               wr, wz, wn, bir, biz, bin_, bhr, bhz, bhn,
               gat_W, atti, attj, gat_b.reshape(1, DIM),
               gnn_g.reshape(1, DIM), gnn_beta.reshape(1, DIM),
               bno_g.reshape(1, DIM), bno_beta.reshape(1, DIM),
               out_W.reshape(1, DIM), out_b.reshape(1, 1))

    return (out.reshape(B, N), xr.reshape(B, N, W), z.reshape(B, N, W))
